# baseline (speedup 1.0000x reference)
import jax
import jax.numpy as jnp
from jax.experimental import pallas as pl
from jax.experimental.pallas import tpu as pltpu

HID = 1024
MAXL = 8192
BATCH = 4
S = 512
NB = MAXL // S


def _pe_add_kernel(k_ref, vel_ref, pe_lo_ref, pe_hi_ref, x_ref, o_ref):
    i = pl.program_id(0)
    k = k_ref[i]
    v = jnp.clip(vel_ref[0, 0], 0.0, 0.99)
    gamma = 1.0 / jnp.sqrt(1.0 - v * v)
    pos = (jax.lax.broadcasted_iota(jnp.int32, (S, 1), 0) + i * S).astype(jnp.float32)
    rel = jnp.clip(pos / gamma, 0.0, float(MAXL - 1))
    rfl = jnp.floor(rel)
    wh = rel - rfl
    wl = 1.0 - wh
    lo = rfl.astype(jnp.int32) - k * S
    lo = jnp.clip(lo, 0, 2 * S - 1)
    hi = jnp.minimum(lo + 1, 2 * S - 1)
    col = jax.lax.broadcasted_iota(jnp.int32, (S, 2 * S), 1)
    w = jnp.where(col == lo, wl, 0.0) + jnp.where(col == hi, wh, 0.0)
    slab = jnp.concatenate([pe_lo_ref[...], pe_hi_ref[...]], axis=0)
    pe = jax.lax.dot_general(
        w.astype(jnp.bfloat16), slab.astype(jnp.bfloat16),
        (((1,), (0,)), ((), ())), preferred_element_type=jnp.float32)
    o_ref[...] = x_ref[...] + pe[None, :, :]


def kernel(x, velocity, pe_base):
    pe2d = pe_base[0]
    v = jnp.clip(velocity[0], 0.0, 0.99)
    gamma = 1.0 / jnp.sqrt(1.0 - v * v)
    p0 = jnp.arange(NB, dtype=jnp.float32) * S
    b = jnp.floor(jnp.clip(p0 / gamma, 0.0, float(MAXL - 1)))
    k_arr = jnp.clip(jnp.floor((b - 4.0) / S), 0.0, float(NB - 2)).astype(jnp.int32)
    vel2d = velocity.reshape(1, 1)

    grid_spec = pltpu.PrefetchScalarGridSpec(
        num_scalar_prefetch=1,
        grid=(NB,),
        in_specs=[
            pl.BlockSpec((1, 1), lambda i, k: (0, 0)),
            pl.BlockSpec((S, HID), lambda i, k: (k[i], 0)),
            pl.BlockSpec((S, HID), lambda i, k: (k[i] + 1, 0)),
            pl.BlockSpec((BATCH, S, HID), lambda i, k: (0, i, 0)),
        ],
        out_specs=pl.BlockSpec((BATCH, S, HID), lambda i, k: (0, i, 0)),
    )
    return pl.pallas_call(
        _pe_add_kernel,
        grid_spec=grid_spec,
        out_shape=jax.ShapeDtypeStruct(x.shape, x.dtype),
        compiler_params=pltpu.CompilerParams(
            dimension_semantics=("parallel",)),
    )(k_arr, vel2d, pe2d, pe2d, x)


# speedup vs baseline: 1.0443x; 1.0443x over previous
import jax
import jax.numpy as jnp
from jax.experimental import pallas as pl
from jax.experimental.pallas import tpu as pltpu

HID = 1024
MAXL = 8192
BATCH = 4
S = 512
NB = MAXL // S
R = S + 16


def _pe_add_kernel(b_ref, vel_ref, pe_hbm, x_ref, o_ref, slab_ref, sem_ref):
    i = pl.program_id(0)

    def start_copy(step, slot):
        pltpu.make_async_copy(
            pe_hbm.at[pl.ds(pl.multiple_of(b_ref[step], 8), R), :],
            slab_ref.at[slot], sem_ref.at[slot]).start()

    @pl.when(i == 0)
    def _():
        start_copy(0, 0)

    @pl.when(i + 1 < NB)
    def _():
        start_copy(i + 1, (i + 1) % 2)

    base = b_ref[i]
    pltpu.make_async_copy(
        pe_hbm.at[pl.ds(pl.multiple_of(base, 8), R), :],
        slab_ref.at[i % 2], sem_ref.at[i % 2]).wait()

    v = jnp.clip(vel_ref[0, 0], 0.0, 0.99)
    gamma = 1.0 / jnp.sqrt(1.0 - v * v)
    pos = (jax.lax.broadcasted_iota(jnp.int32, (S, 1), 0) + i * S).astype(jnp.float32)
    rel = jnp.clip(pos / gamma, 0.0, float(MAXL - 1))
    rfl = jnp.floor(rel)
    wh = rel - rfl
    wl = 1.0 - wh
    lo = rfl.astype(jnp.int32) - base
    lo = jnp.clip(lo, 0, R - 1)
    hi = jnp.minimum(lo + 1, R - 1)
    col = jax.lax.broadcasted_iota(jnp.int32, (S, R), 1)
    w = jnp.where(col == lo, wl, 0.0) + jnp.where(col == hi, wh, 0.0)
    pe = jax.lax.dot_general(
        w.astype(jnp.bfloat16), slab_ref[i % 2].astype(jnp.bfloat16),
        (((1,), (0,)), ((), ())), preferred_element_type=jnp.float32)
    o_ref[...] = x_ref[...] + pe[None, :, :]


def kernel(x, velocity, pe_base):
    pe2d = pe_base[0]
    v = jnp.clip(velocity[0], 0.0, 0.99)
    gamma = 1.0 / jnp.sqrt(1.0 - v * v)
    p0 = jnp.arange(NB, dtype=jnp.float32) * S
    b = jnp.floor(jnp.clip(p0 / gamma, 0.0, float(MAXL - 1)))
    b_arr = jnp.clip(jnp.floor((b - 4.0) / 8.0) * 8.0, 0.0,
                     float(MAXL - R)).astype(jnp.int32)
    vel2d = velocity.reshape(1, 1)

    grid_spec = pltpu.PrefetchScalarGridSpec(
        num_scalar_prefetch=1,
        grid=(NB,),
        in_specs=[
            pl.BlockSpec((1, 1), lambda i, bb: (0, 0)),
            pl.BlockSpec(memory_space=pl.ANY),
            pl.BlockSpec((BATCH, S, HID), lambda i, bb: (0, i, 0)),
        ],
        out_specs=pl.BlockSpec((BATCH, S, HID), lambda i, bb: (0, i, 0)),
        scratch_shapes=[
            pltpu.VMEM((2, R, HID), jnp.float32),
            pltpu.SemaphoreType.DMA((2,)),
        ],
    )
    return pl.pallas_call(
        _pe_add_kernel,
        grid_spec=grid_spec,
        out_shape=jax.ShapeDtypeStruct(x.shape, x.dtype),
        compiler_params=pltpu.CompilerParams(
            dimension_semantics=("arbitrary",)),
    )(b_arr, vel2d, pe2d, x)
